# baseline (speedup 1.0000x reference)
import jax
import jax.numpy as jnp
from jax import lax
from jax.experimental import pallas as pl
from jax.experimental.pallas import tpu as pltpu
from jax.experimental.pallas import tpu_sc as plsc

N = 10000
E = 320000
D = 128
H = 8
C = 16
HC = H * C
TW = HC + 16

NB = 400
NBLK = N // NB

ECH = 80
NCHUNK = E // ECH
NWORK = 32
_BASE_CH = NCHUNK // NWORK
NPAD = 10240
RPS = NPAD // 16

_HI = lax.Precision.HIGHEST


def _prep_body(x_ref, wp_ref, as_ref, ad_ref, tsrc_ref, tdst_ref, pmax_ref):
    hc = jnp.dot(x_ref[...], wp_ref[...], precision=_HI)
    a_s = jnp.dot(hc, as_ref[...], precision=_HI)
    a_d = jnp.dot(hc, ad_ref[...], precision=_HI)
    zeros8 = jnp.zeros((NB, 8), jnp.float32)
    tsrc_ref[...] = jnp.concatenate([hc, a_s, zeros8], axis=1)
    tdst_ref[...] = jnp.concatenate([a_d, zeros8], axis=1)
    pmax_ref[...] = jnp.concatenate(
        [jnp.max(a_s, axis=0), jnp.max(a_d, axis=0)]).reshape(1, 1, 16)


def _prep(x, Wp, A_s, A_d):
    return pl.pallas_call(
        _prep_body,
        grid=(NBLK,),
        in_specs=[
            pl.BlockSpec((NB, D), lambda i: (i, 0)),
            pl.BlockSpec((D, HC), lambda i: (0, 0)),
            pl.BlockSpec((HC, 8), lambda i: (0, 0)),
            pl.BlockSpec((HC, 8), lambda i: (0, 0)),
        ],
        out_specs=[
            pl.BlockSpec((NB, TW), lambda i: (i, 0)),
            pl.BlockSpec((NB, 16), lambda i: (i, 0)),
            pl.BlockSpec((1, 1, 16), lambda i: (i, 0, 0)),
        ],
        out_shape=[
            jax.ShapeDtypeStruct((N, TW), jnp.float32),
            jax.ShapeDtypeStruct((N, 16), jnp.float32),
            jax.ShapeDtypeStruct((NBLK, 1, 16), jnp.float32),
        ],
    )(x, Wp, A_s, A_d)


def _sc_body(tsrc_hbm, tdst_hbm, src_hbm, dst_hbm, m_hbm, out_hbm,
             is0, id0, rows0, drows0, is1, id1, rows1, drows1,
             mvec, acc, gsem0, gsem1):
    cid = lax.axis_index("c")
    sid = lax.axis_index("s")
    wid = sid * 2 + cid

    bufs = ((is0, id0, rows0, drows0, gsem0),
            (is1, id1, rows1, drows1, gsem1))

    def fire(c, b):
        is_, id_, rows_, drows_, sem = bufs[b]
        off = c * ECH
        pltpu.sync_copy(src_hbm.at[pl.ds(off, ECH)], is_)
        pltpu.sync_copy(dst_hbm.at[pl.ds(off, ECH)], id_)
        pltpu.async_copy(tsrc_hbm.at[is_], rows_, sem)
        pltpu.async_copy(tdst_hbm.at[id_], drows_, sem)

    def drain(b):
        is_, id_, rows_, drows_, sem = bufs[b]
        pltpu.make_async_copy(tsrc_hbm.at[is_], rows_, sem).wait()
        pltpu.make_async_copy(tdst_hbm.at[id_], drows_, sem).wait()

    zero16 = jnp.zeros((16,), jnp.float32)

    @pl.loop(0, ECH)
    def _(r):
        for k in range(TW // 16):
            rows0[r, pl.ds(16 * k, 16)] = zero16

    @pl.loop(0, RPS // ECH)
    def _(z):
        pltpu.sync_copy(rows0.at[pl.ds(0, ECH)],
                        acc.at[pl.ds(sid * RPS + z * ECH, ECH)])

    plsc.subcore_barrier()

    pltpu.sync_copy(m_hbm, mvec)
    m = mvec[...]
    pat = lax.rem(lax.iota(jnp.int32, 16), jnp.full((16,), 8, jnp.int32))
    colv = pat + jnp.full((16,), HC, jnp.int32)

    def compute(b):
        rows_, drows_ = bufs[b][2], bufs[b][3]

        @plsc.parallel_loop(0, ECH, unroll=8)
        def _(e):
            a_s = rows_[e, pl.ds(HC, 16)]
            a_d = drows_[e, pl.ds(0, 16)]
            t = a_s + a_d
            lrelu = jnp.maximum(t, 0.2 * t)
            wv = jnp.exp(lrelu - m)
            rows_[e, pl.ds(HC, 16)] = wv
            rowv = jnp.full((16,), e, jnp.int32)
            wt = plsc.load_gather(rows_, [rowv, colv])
            for k in range(H):
                sl = pl.ds(16 * k, 16)
                rows_[e, sl] = rows_[e, sl] * wt

    def scatter(b):
        id_, rows_ = bufs[b][1], bufs[b][2]
        pltpu.sync_copy(rows_, acc.at[id_], add=True)

    fire(wid, 0)
    fire(wid + NWORK, 1)

    @pl.loop(0, _BASE_CH // 2)
    def _(t):
        k0 = 2 * t
        for b in range(2):
            k = k0 + b
            drain(b)
            compute(b)
            scatter(b)

            @pl.when(k + 2 < _BASE_CH)
            def _():
                fire(wid + NWORK * (k + 2), b)

    drain(0)
    compute(0)
    scatter(0)

    plsc.subcore_barrier()

    @pl.loop(0, 5)
    def _(z):
        r0 = sid * RPS + z * (RPS // 5)
        pltpu.sync_copy(acc.at[pl.ds(r0, RPS // 5)],
                        out_hbm.at[cid, pl.ds(r0, RPS // 5)])


_SC_CP = pltpu.CompilerParams(needs_layout_passes=False,
                              use_tc_tiling_on_sc=False)


def _sc_edges(tsrc, tdst, src, dst, m16):
    return pl.kernel(
        _sc_body,
        compiler_params=_SC_CP,
        out_type=jax.ShapeDtypeStruct((2, NPAD, TW), jnp.float32),
        mesh=plsc.VectorSubcoreMesh(core_axis_name="c", subcore_axis_name="s"),
        scratch_types=[
            pltpu.VMEM((ECH,), jnp.int32),
            pltpu.VMEM((ECH,), jnp.int32),
            pltpu.VMEM((ECH, TW), jnp.float32),
            pltpu.VMEM((ECH, 16), jnp.float32),
            pltpu.VMEM((ECH,), jnp.int32),
            pltpu.VMEM((ECH,), jnp.int32),
            pltpu.VMEM((ECH, TW), jnp.float32),
            pltpu.VMEM((ECH, 16), jnp.float32),
            pltpu.VMEM((16,), jnp.float32),
            pltpu.VMEM_SHARED((NPAD, TW), jnp.float32),
            pltpu.SemaphoreType.DMA,
            pltpu.SemaphoreType.DMA,
        ],
    )(tsrc, tdst, src, dst, m16)


def _final_body(p_ref, tsrc_ref, tdst_ref, m_ref, b_ref, perm_ref, o_ref):
    tsrc = tsrc_ref[...]
    asrc = tsrc[:, HC:HC + 8]
    adst = tdst_ref[...][:, :8]
    t = asrc + adst
    lrelu = jnp.maximum(t, 0.2 * t)
    wself = jnp.exp(lrelu - m_ref[0, :8][None, :])
    p0 = p_ref[0]
    p1 = p_ref[1]
    num = p0[:, :HC] + p1[:, :HC] + tsrc[:, :HC] * jnp.tile(wself, (1, C))
    den = p0[:, HC:HC + 8] + p1[:, HC:HC + 8] + wself + 1e-16
    outc = num / jnp.tile(den, (1, C))
    o_ref[...] = (jnp.dot(outc, perm_ref[...], precision=_HI)
                  + b_ref[0][None, :])


def _final(p, tsrc, tdst, m16, bias, P):
    return pl.pallas_call(
        _final_body,
        grid=(NBLK,),
        in_specs=[
            pl.BlockSpec((2, NB, TW), lambda i: (0, i, 0)),
            pl.BlockSpec((NB, TW), lambda i: (i, 0)),
            pl.BlockSpec((NB, 16), lambda i: (i, 0)),
            pl.BlockSpec((1, 16), lambda i: (0, 0)),
            pl.BlockSpec((1, HC), lambda i: (0, 0)),
            pl.BlockSpec((HC, HC), lambda i: (0, 0)),
        ],
        out_specs=pl.BlockSpec((NB, HC), lambda i: (i, 0)),
        out_shape=jax.ShapeDtypeStruct((N, HC), jnp.float32),
    )(p, tsrc, tdst, m16, bias, P)


def kernel(x, edge_index, W, att_src, att_dst, bias):
    idx = jnp.arange(HC, dtype=jnp.int32)
    cmajor_of = (idx % 8) * C + idx // 8
    Wp = W[:, cmajor_of]
    onehot = jax.nn.one_hot(idx % 8, 8, dtype=jnp.float32)
    A_s = att_src.reshape(H, C).T.reshape(HC, 1) * onehot
    A_d = att_dst.reshape(H, C).T.reshape(HC, 1) * onehot
    P = jax.nn.one_hot(cmajor_of, HC, dtype=jnp.float32)

    tsrc, tdst, pmax = _prep(x, Wp, A_s, A_d)
    pm = pmax.reshape(NBLK, 16)
    m8 = jnp.max(pm[:, :8], axis=0) + jnp.max(pm[:, 8:], axis=0)
    m16 = jnp.concatenate([m8, jnp.full((8,), 1e30, jnp.float32)])
    partials = _sc_edges(tsrc, tdst, edge_index[0], edge_index[1], m16)
    return _final(partials, tsrc, tdst,
                  m16.reshape(1, 16), bias.reshape(1, HC), P)


# speedup vs baseline: 1.2984x; 1.2984x over previous
import jax
import jax.numpy as jnp
from jax import lax
from jax.experimental import pallas as pl
from jax.experimental.pallas import tpu as pltpu
from jax.experimental.pallas import tpu_sc as plsc

N = 10000
E = 320000
D = 128
H = 8
C = 16
HC = H * C
TW = HC + 16

NB = 400
NBLK = N // NB

ECH = 80
NCHUNK = E // ECH
NWORK = 32
_BASE_CH = NCHUNK // NWORK
NPAD = 10240
RPS = NPAD // 16

_HI = lax.Precision.HIGHEST


def _prep_body(x_ref, wp_ref, as_ref, ad_ref, tsrc_ref, tdst_ref, pmax_ref):
    hc = jnp.dot(x_ref[...], wp_ref[...], precision=_HI)
    a_s = jnp.dot(hc, as_ref[...], precision=_HI)
    a_d = jnp.dot(hc, ad_ref[...], precision=_HI)
    zeros8 = jnp.zeros((NB, 8), jnp.float32)
    tsrc_ref[...] = jnp.concatenate([hc, a_s, zeros8], axis=1)
    tdst_ref[...] = jnp.concatenate([a_d, zeros8], axis=1)
    pmax_ref[...] = jnp.concatenate(
        [jnp.max(a_s, axis=0), jnp.max(a_d, axis=0)]).reshape(1, 1, 16)


def _prep(x, Wp, A_s, A_d):
    return pl.pallas_call(
        _prep_body,
        grid=(NBLK,),
        in_specs=[
            pl.BlockSpec((NB, D), lambda i: (i, 0)),
            pl.BlockSpec((D, HC), lambda i: (0, 0)),
            pl.BlockSpec((HC, 8), lambda i: (0, 0)),
            pl.BlockSpec((HC, 8), lambda i: (0, 0)),
        ],
        out_specs=[
            pl.BlockSpec((NB, TW), lambda i: (i, 0)),
            pl.BlockSpec((NB, 16), lambda i: (i, 0)),
            pl.BlockSpec((1, 1, 16), lambda i: (i, 0, 0)),
        ],
        out_shape=[
            jax.ShapeDtypeStruct((N, TW), jnp.float32),
            jax.ShapeDtypeStruct((N, 16), jnp.float32),
            jax.ShapeDtypeStruct((NBLK, 1, 16), jnp.float32),
        ],
    )(x, Wp, A_s, A_d)


def _sc_body(tsrc_hbm, tdst_hbm, src2_hbm, dst2_hbm, m_hbm, out_hbm,
             is00, id00, is01, id01, is10, id10, is11, id11,
             rows0, drows0, rows1, drows1, mvec, acc,
             gsem0, gsem1, isem00, isem01, isem10, isem11):
    cid = lax.axis_index("c")
    sid = lax.axis_index("s")
    wid = sid * 2 + cid
    base = wid * _BASE_CH

    rowb = (rows0, rows1)
    drowb = (drows0, drows1)
    gsem = (gsem0, gsem1)
    idx = (((is00, id00, isem00), (is01, id01, isem01)),
           ((is10, id10, isem10), (is11, id11, isem11)))

    def fire_idx(k, b, q):
        is_, id_, sem = idx[b][q]
        pltpu.async_copy(src2_hbm.at[base + k], is_, sem)
        pltpu.async_copy(dst2_hbm.at[base + k], id_, sem)

    def fire_gather(b, q):
        is_, id_, sem = idx[b][q]
        pltpu.make_async_copy(src2_hbm.at[base], is_, sem).wait()
        pltpu.make_async_copy(dst2_hbm.at[base], id_, sem).wait()
        pltpu.async_copy(tsrc_hbm.at[is_], rowb[b], gsem[b])
        pltpu.async_copy(tdst_hbm.at[id_], drowb[b], gsem[b])

    def drain_gather(b, q):
        is_, id_, _ = idx[b][q]
        pltpu.make_async_copy(tsrc_hbm.at[is_], rowb[b], gsem[b]).wait()
        pltpu.make_async_copy(tdst_hbm.at[id_], drowb[b], gsem[b]).wait()

    zero16 = jnp.zeros((16,), jnp.float32)

    @pl.loop(0, ECH)
    def _(r):
        for k in range(TW // 16):
            rows0[r, pl.ds(16 * k, 16)] = zero16

    @pl.loop(0, RPS // ECH)
    def _(z):
        pltpu.sync_copy(rows0.at[pl.ds(0, ECH)],
                        acc.at[pl.ds(sid * RPS + z * ECH, ECH)])

    plsc.subcore_barrier()

    pltpu.sync_copy(m_hbm, mvec)
    m = mvec[...]
    pat = lax.rem(lax.iota(jnp.int32, 16), jnp.full((16,), 8, jnp.int32))
    colv = pat + jnp.full((16,), HC, jnp.int32)

    def compute(b):
        rows_, drows_ = rowb[b], drowb[b]

        @plsc.parallel_loop(0, ECH, unroll=8)
        def _(e):
            a_s = rows_[e, pl.ds(HC, 16)]
            a_d = drows_[e, pl.ds(0, 16)]
            t = a_s + a_d
            lrelu = jnp.maximum(t, 0.2 * t)
            wv = jnp.exp(lrelu - m)
            rows_[e, pl.ds(HC, 16)] = wv
            rowv = jnp.full((16,), e, jnp.int32)
            wt = plsc.load_gather(rows_, [rowv, colv])
            for k in range(H):
                sl = pl.ds(16 * k, 16)
                rows_[e, sl] = rows_[e, sl] * wt

    def scatter(b, q):
        pltpu.sync_copy(rowb[b], acc.at[idx[b][q][1]], add=True)

    for k in range(4):
        fire_idx(k, k % 2, (k // 2) % 2)
    for k in range(2):
        fire_gather(k % 2, 0)

    @pl.loop(0, (_BASE_CH - 1) // 4)
    def _(u):
        for r in range(4):
            k = 4 * u + r
            b = r % 2
            q = r // 2
            drain_gather(b, q)
            compute(b)
            scatter(b, q)

            @pl.when(k + 4 < _BASE_CH)
            def _():
                fire_idx(k + 4, b, q)

            @pl.when(k + 2 < _BASE_CH)
            def _():
                fire_gather(b, 1 - q)

    drain_gather(0, 0)
    compute(0)
    scatter(0, 0)

    plsc.subcore_barrier()

    @pl.loop(0, 5)
    def _(z):
        r0 = sid * RPS + z * (RPS // 5)
        pltpu.sync_copy(acc.at[pl.ds(r0, RPS // 5)],
                        out_hbm.at[cid, pl.ds(r0, RPS // 5)])


_SC_CP = pltpu.CompilerParams(needs_layout_passes=False,
                              use_tc_tiling_on_sc=False)


def _sc_edges(tsrc, tdst, src2, dst2, m16):
    return pl.kernel(
        _sc_body,
        compiler_params=_SC_CP,
        out_type=jax.ShapeDtypeStruct((2, NPAD, TW), jnp.float32),
        mesh=plsc.VectorSubcoreMesh(core_axis_name="c", subcore_axis_name="s"),
        scratch_types=(
            [pltpu.VMEM((ECH,), jnp.int32)] * 8
            + [
                pltpu.VMEM((ECH, TW), jnp.float32),
                pltpu.VMEM((ECH, 16), jnp.float32),
                pltpu.VMEM((ECH, TW), jnp.float32),
                pltpu.VMEM((ECH, 16), jnp.float32),
                pltpu.VMEM((16,), jnp.float32),
                pltpu.VMEM_SHARED((NPAD, TW), jnp.float32),
            ]
            + [pltpu.SemaphoreType.DMA] * 6
        ),
    )(tsrc, tdst, src2, dst2, m16)


def _final_body(p_ref, tsrc_ref, tdst_ref, m_ref, b_ref, perm_ref, o_ref):
    tsrc = tsrc_ref[...]
    asrc = tsrc[:, HC:HC + 8]
    adst = tdst_ref[...][:, :8]
    t = asrc + adst
    lrelu = jnp.maximum(t, 0.2 * t)
    wself = jnp.exp(lrelu - m_ref[0, :8][None, :])
    p0 = p_ref[0]
    p1 = p_ref[1]
    num = p0[:, :HC] + p1[:, :HC] + tsrc[:, :HC] * jnp.tile(wself, (1, C))
    den = p0[:, HC:HC + 8] + p1[:, HC:HC + 8] + wself + 1e-16
    outc = num / jnp.tile(den, (1, C))
    o_ref[...] = jnp.dot(outc, perm_ref[...]) + b_ref[0][None, :]


def _final(p, tsrc, tdst, m16, bias, P):
    return pl.pallas_call(
        _final_body,
        grid=(NBLK,),
        in_specs=[
            pl.BlockSpec((2, NB, TW), lambda i: (0, i, 0)),
            pl.BlockSpec((NB, TW), lambda i: (i, 0)),
            pl.BlockSpec((NB, 16), lambda i: (i, 0)),
            pl.BlockSpec((1, 16), lambda i: (0, 0)),
            pl.BlockSpec((1, HC), lambda i: (0, 0)),
            pl.BlockSpec((HC, HC), lambda i: (0, 0)),
        ],
        out_specs=pl.BlockSpec((NB, HC), lambda i: (i, 0)),
        out_shape=jax.ShapeDtypeStruct((N, HC), jnp.float32),
    )(p, tsrc, tdst, m16, bias, P)


def kernel(x, edge_index, W, att_src, att_dst, bias):
    idx = jnp.arange(HC, dtype=jnp.int32)
    cmajor_of = (idx % 8) * C + idx // 8
    Wp = W[:, cmajor_of]
    onehot = jax.nn.one_hot(idx % 8, 8, dtype=jnp.float32)
    A_s = att_src.reshape(H, C).T.reshape(HC, 1) * onehot
    A_d = att_dst.reshape(H, C).T.reshape(HC, 1) * onehot
    P = jax.nn.one_hot(cmajor_of, HC, dtype=jnp.float32)

    tsrc, tdst, pmax = _prep(x, Wp, A_s, A_d)
    pm = pmax.reshape(NBLK, 16)
    m8 = jnp.max(pm[:, :8], axis=0) + jnp.max(pm[:, 8:], axis=0)
    m16 = jnp.concatenate([m8, jnp.full((8,), 1e30, jnp.float32)])
    src2 = edge_index[0].reshape(NCHUNK, ECH)
    dst2 = edge_index[1].reshape(NCHUNK, ECH)
    partials = _sc_edges(tsrc, tdst, src2, dst2, m16)
    return _final(partials, tsrc, tdst,
                  m16.reshape(1, 16), bias.reshape(1, HC), P)


# speedup vs baseline: 1.5242x; 1.1739x over previous
import jax
import jax.numpy as jnp
from jax import lax
from jax.experimental import pallas as pl
from jax.experimental.pallas import tpu as pltpu
from jax.experimental.pallas import tpu_sc as plsc

N = 10000
E = 320000
D = 128
H = 8
C = 16
HC = H * C
TW = HC + 16

NB = 2000
NBLK = N // NB

ECH = 80
NCHUNK = E // ECH
NWORK = 32
_BASE_CH = NCHUNK // NWORK
NPAD = 10240
RPS = NPAD // 16

_HI = lax.Precision.HIGHEST


def _prep_body(x_ref, wp_ref, as_ref, ad_ref, tsrc_ref, tdst_ref,
               pmax_ref, m16_ref):
    hc = jnp.dot(x_ref[...], wp_ref[...], precision=_HI)
    a_s = jnp.dot(hc, as_ref[...], precision=_HI)
    a_d = jnp.dot(hc, ad_ref[...], precision=_HI)
    zeros8 = jnp.zeros((NB, 8), jnp.float32)
    tsrc_ref[...] = jnp.concatenate([hc, a_s, zeros8], axis=1)
    tdst_ref[...] = jnp.concatenate([a_d, zeros8], axis=1)
    cur = jnp.broadcast_to(
        jnp.concatenate([jnp.max(a_s, axis=0), jnp.max(a_d, axis=0)])[None, :],
        (8, 16))
    i = pl.program_id(0)

    @pl.when(i == 0)
    def _():
        pmax_ref[...] = cur

    @pl.when(i > 0)
    def _():
        pmax_ref[...] = jnp.maximum(pmax_ref[...], cur)

    @pl.when(i == NBLK - 1)
    def _():
        pm = pmax_ref[...]
        tot = pm + jnp.roll(pm, -8, axis=1)
        lane = lax.broadcasted_iota(jnp.int32, (8, 16), 1)
        m16_ref[...] = jnp.where(lane < 8, tot, 1e30)


def _prep(x, Wp, A_s, A_d):
    return pl.pallas_call(
        _prep_body,
        grid=(NBLK,),
        in_specs=[
            pl.BlockSpec((NB, D), lambda i: (i, 0)),
            pl.BlockSpec((D, HC), lambda i: (0, 0)),
            pl.BlockSpec((HC, 8), lambda i: (0, 0)),
            pl.BlockSpec((HC, 8), lambda i: (0, 0)),
        ],
        out_specs=[
            pl.BlockSpec((NB, TW), lambda i: (i, 0)),
            pl.BlockSpec((NB, 16), lambda i: (i, 0)),
            pl.BlockSpec((8, 16), lambda i: (0, 0)),
            pl.BlockSpec((8, 16), lambda i: (0, 0)),
        ],
        out_shape=[
            jax.ShapeDtypeStruct((N, TW), jnp.float32),
            jax.ShapeDtypeStruct((N, 16), jnp.float32),
            jax.ShapeDtypeStruct((8, 16), jnp.float32),
            jax.ShapeDtypeStruct((8, 16), jnp.float32),
        ],
    )(x, Wp, A_s, A_d)


def _sc_body(tsrc_hbm, tdst_hbm, src2_hbm, dst2_hbm, m_hbm, out_hbm,
             is00, id00, is01, id01, is10, id10, is11, id11,
             rows0, drows0, rows1, drows1, mvec, acc,
             gsem0, gsem1, isem00, isem01, isem10, isem11):
    cid = lax.axis_index("c")
    sid = lax.axis_index("s")
    wid = sid * 2 + cid
    base = wid * _BASE_CH

    rowb = (rows0, rows1)
    drowb = (drows0, drows1)
    gsem = (gsem0, gsem1)
    idx = (((is00, id00, isem00), (is01, id01, isem01)),
           ((is10, id10, isem10), (is11, id11, isem11)))

    def fire_idx(k, b, q):
        is_, id_, sem = idx[b][q]
        pltpu.async_copy(src2_hbm.at[base + k], is_, sem)
        pltpu.async_copy(dst2_hbm.at[base + k], id_, sem)

    def fire_gather(b, q):
        is_, id_, sem = idx[b][q]
        pltpu.make_async_copy(src2_hbm.at[base], is_, sem).wait()
        pltpu.make_async_copy(dst2_hbm.at[base], id_, sem).wait()
        pltpu.async_copy(tsrc_hbm.at[is_], rowb[b], gsem[b])
        pltpu.async_copy(tdst_hbm.at[id_], drowb[b], gsem[b])

    def drain_gather(b, q):
        is_, id_, _ = idx[b][q]
        pltpu.make_async_copy(tsrc_hbm.at[is_], rowb[b], gsem[b]).wait()
        pltpu.make_async_copy(tdst_hbm.at[id_], drowb[b], gsem[b]).wait()

    zero16 = jnp.zeros((16,), jnp.float32)

    @pl.loop(0, ECH)
    def _(r):
        for k in range(TW // 16):
            rows0[r, pl.ds(16 * k, 16)] = zero16

    @pl.loop(0, RPS // ECH)
    def _(z):
        pltpu.sync_copy(rows0.at[pl.ds(0, ECH)],
                        acc.at[pl.ds(sid * RPS + z * ECH, ECH)])

    plsc.subcore_barrier()

    pltpu.sync_copy(m_hbm.at[0], mvec)
    m = mvec[...]
    pat = lax.rem(lax.iota(jnp.int32, 16), jnp.full((16,), 8, jnp.int32))
    colv = pat + jnp.full((16,), HC, jnp.int32)

    def compute(b):
        rows_, drows_ = rowb[b], drowb[b]

        @plsc.parallel_loop(0, ECH, unroll=8)
        def _(e):
            a_s = rows_[e, pl.ds(HC, 16)]
            a_d = drows_[e, pl.ds(0, 16)]
            t = a_s + a_d
            lrelu = jnp.maximum(t, 0.2 * t)
            wv = jnp.exp(lrelu - m)
            rows_[e, pl.ds(HC, 16)] = wv
            rowv = jnp.full((16,), e, jnp.int32)
            wt = plsc.load_gather(rows_, [rowv, colv])
            for k in range(H):
                sl = pl.ds(16 * k, 16)
                rows_[e, sl] = rows_[e, sl] * wt

    def scatter(b, q):
        pltpu.sync_copy(rowb[b], acc.at[idx[b][q][1]], add=True)

    for k in range(4):
        fire_idx(k, k % 2, (k // 2) % 2)
    for k in range(2):
        fire_gather(k % 2, 0)

    @pl.loop(0, (_BASE_CH - 1) // 4)
    def _(u):
        for r in range(4):
            k = 4 * u + r
            b = r % 2
            q = r // 2
            drain_gather(b, q)
            compute(b)
            scatter(b, q)

            @pl.when(k + 4 < _BASE_CH)
            def _():
                fire_idx(k + 4, b, q)

            @pl.when(k + 2 < _BASE_CH)
            def _():
                fire_gather(b, 1 - q)

    drain_gather(0, 0)
    compute(0)
    scatter(0, 0)

    plsc.subcore_barrier()

    @pl.loop(0, 5)
    def _(z):
        r0 = sid * RPS + z * (RPS // 5)
        pltpu.sync_copy(acc.at[pl.ds(r0, RPS // 5)],
                        out_hbm.at[cid, pl.ds(r0, RPS // 5)])


_SC_CP = pltpu.CompilerParams(needs_layout_passes=False,
                              use_tc_tiling_on_sc=False)


def _sc_edges(tsrc, tdst, src2, dst2, m16):
    return pl.kernel(
        _sc_body,
        compiler_params=_SC_CP,
        out_type=jax.ShapeDtypeStruct((2, NPAD, TW), jnp.float32),
        mesh=plsc.VectorSubcoreMesh(core_axis_name="c", subcore_axis_name="s"),
        scratch_types=(
            [pltpu.VMEM((ECH,), jnp.int32)] * 8
            + [
                pltpu.VMEM((ECH, TW), jnp.float32),
                pltpu.VMEM((ECH, 16), jnp.float32),
                pltpu.VMEM((ECH, TW), jnp.float32),
                pltpu.VMEM((ECH, 16), jnp.float32),
                pltpu.VMEM((16,), jnp.float32),
                pltpu.VMEM_SHARED((NPAD, TW), jnp.float32),
            ]
            + [pltpu.SemaphoreType.DMA] * 6
        ),
    )(tsrc, tdst, src2, dst2, m16)


def _final_body(p_ref, tsrc_ref, tdst_ref, m_ref, b_ref, perm_ref, t8_ref,
                o_ref):
    tsrc = tsrc_ref[...]
    asrc = tsrc[:, HC:HC + 8]
    adst = tdst_ref[...][:, :8]
    t = asrc + adst
    lrelu = jnp.maximum(t, 0.2 * t)
    wself = jnp.exp(lrelu - m_ref[0, :8][None, :])
    p0 = p_ref[0]
    p1 = p_ref[1]
    t8 = t8_ref[...]
    num = p0[:, :HC] + p1[:, :HC] + tsrc[:, :HC] * jnp.dot(wself, t8)
    rden = 1.0 / (p0[:, HC:HC + 8] + p1[:, HC:HC + 8] + wself + 1e-16)
    outc = num * jnp.dot(rden, t8)
    o_ref[...] = jnp.dot(outc, perm_ref[...]) + b_ref[0][None, :]


def _final(p, tsrc, tdst, m16, bias, P, T8):
    return pl.pallas_call(
        _final_body,
        grid=(NBLK,),
        in_specs=[
            pl.BlockSpec((2, NB, TW), lambda i: (0, i, 0)),
            pl.BlockSpec((NB, TW), lambda i: (i, 0)),
            pl.BlockSpec((NB, 16), lambda i: (i, 0)),
            pl.BlockSpec((8, 16), lambda i: (0, 0)),
            pl.BlockSpec((1, HC), lambda i: (0, 0)),
            pl.BlockSpec((HC, HC), lambda i: (0, 0)),
            pl.BlockSpec((8, HC), lambda i: (0, 0)),
        ],
        out_specs=pl.BlockSpec((NB, HC), lambda i: (i, 0)),
        out_shape=jax.ShapeDtypeStruct((N, HC), jnp.float32),
    )(p, tsrc, tdst, m16, bias, P, T8)


def kernel(x, edge_index, W, att_src, att_dst, bias):
    idx = jnp.arange(HC, dtype=jnp.int32)
    cmajor_of = (idx % 8) * C + idx // 8
    Wp = W[:, cmajor_of]
    onehot = jax.nn.one_hot(idx % 8, 8, dtype=jnp.float32)
    A_s = att_src.reshape(H, C).T.reshape(HC, 1) * onehot
    A_d = att_dst.reshape(H, C).T.reshape(HC, 1) * onehot
    P = jax.nn.one_hot(cmajor_of, HC, dtype=jnp.float32)
    T8 = jax.nn.one_hot(idx % 8, 8, dtype=jnp.float32).T

    tsrc, tdst, _, m16 = _prep(x, Wp, A_s, A_d)
    src2 = edge_index[0].reshape(NCHUNK, ECH)
    dst2 = edge_index[1].reshape(NCHUNK, ECH)
    partials = _sc_edges(tsrc, tdst, src2, dst2, m16)
    return _final(partials, tsrc, tdst, m16, bias.reshape(1, HC), P, T8)


# speedup vs baseline: 1.6306x; 1.0698x over previous
import jax
import jax.numpy as jnp
from jax import lax
from jax.experimental import pallas as pl
from jax.experimental.pallas import tpu as pltpu
from jax.experimental.pallas import tpu_sc as plsc

N = 10000
E = 320000
D = 128
H = 8
C = 16
HC = H * C
TW = HC + 16

NB = 2000
NBLK = N // NB

ECH = 80
NCHUNK = E // ECH
NWORK = 32
_BASE_CH = NCHUNK // NWORK
NPAD = 10240
RPS = NPAD // 16

_HI = lax.Precision.HIGHEST


def _prep_body(x_ref, wp_ref, as_ref, ad_ref, tsrc_ref, tdst_ref,
               pmax_ref, m16_ref):
    hc = jnp.dot(x_ref[...], wp_ref[...], precision=_HI)
    a_s = jnp.dot(hc, as_ref[...], precision=_HI)
    a_d = jnp.dot(hc, ad_ref[...], precision=_HI)
    zeros8 = jnp.zeros((NB, 8), jnp.float32)
    tsrc_ref[...] = jnp.concatenate([hc, a_s, zeros8], axis=1)
    tdst_ref[...] = jnp.concatenate([a_d, zeros8], axis=1)
    cur = jnp.broadcast_to(
        jnp.concatenate([jnp.max(a_s, axis=0), jnp.max(a_d, axis=0)])[None, :],
        (8, 16))
    i = pl.program_id(0)

    @pl.when(i == 0)
    def _():
        pmax_ref[...] = cur

    @pl.when(i > 0)
    def _():
        pmax_ref[...] = jnp.maximum(pmax_ref[...], cur)

    @pl.when(i == NBLK - 1)
    def _():
        pm = pmax_ref[...]
        tot = pm + jnp.roll(pm, -8, axis=1)
        lane = lax.broadcasted_iota(jnp.int32, (8, 16), 1)
        m16_ref[...] = jnp.where(lane < 8, tot, 1e30)


def _prep(x, Wp, A_s, A_d):
    return pl.pallas_call(
        _prep_body,
        grid=(NBLK,),
        in_specs=[
            pl.BlockSpec((NB, D), lambda i: (i, 0)),
            pl.BlockSpec((D, HC), lambda i: (0, 0)),
            pl.BlockSpec((HC, 8), lambda i: (0, 0)),
            pl.BlockSpec((HC, 8), lambda i: (0, 0)),
        ],
        out_specs=[
            pl.BlockSpec((NB, TW), lambda i: (i, 0)),
            pl.BlockSpec((NB, 16), lambda i: (i, 0)),
            pl.BlockSpec((8, 16), lambda i: (0, 0)),
            pl.BlockSpec((8, 16), lambda i: (0, 0)),
        ],
        out_shape=[
            jax.ShapeDtypeStruct((N, TW), jnp.float32),
            jax.ShapeDtypeStruct((N, 16), jnp.float32),
            jax.ShapeDtypeStruct((8, 16), jnp.float32),
            jax.ShapeDtypeStruct((8, 16), jnp.float32),
        ],
    )(x, Wp, A_s, A_d)


def _sc_body(tsrc_hbm, tdst_hbm, ei_hbm, m_hbm, outm_hbm, outd_hbm,
             is00, id00, is01, id01, is10, id10, is11, id11,
             rows0, drows0, rows1, drows1, mvec, acc,
             gsem0, gsem1, isem00, isem01, isem10, isem11):
    cid = lax.axis_index("c")
    sid = lax.axis_index("s")
    wid = sid * 2 + cid
    base = wid * _BASE_CH * ECH

    rowb = (rows0, rows1)
    drowb = (drows0, drows1)
    gsem = (gsem0, gsem1)
    idx = (((is00, id00, isem00), (is01, id01, isem01)),
           ((is10, id10, isem10), (is11, id11, isem11)))

    def fire_idx(k, b, q):
        is_, id_, sem = idx[b][q]
        off = base + k * ECH
        pltpu.async_copy(ei_hbm.at[0, pl.ds(off, ECH)], is_, sem)
        pltpu.async_copy(ei_hbm.at[1, pl.ds(off, ECH)], id_, sem)

    def fire_gather(b, q):
        is_, id_, sem = idx[b][q]
        pltpu.make_async_copy(ei_hbm.at[0, pl.ds(0, ECH)], is_, sem).wait()
        pltpu.make_async_copy(ei_hbm.at[1, pl.ds(0, ECH)], id_, sem).wait()
        pltpu.async_copy(tsrc_hbm.at[is_], rowb[b], gsem[b])
        pltpu.async_copy(tdst_hbm.at[id_], drowb[b], gsem[b])

    def drain_gather(b, q):
        is_, id_, _ = idx[b][q]
        pltpu.make_async_copy(tsrc_hbm.at[is_], rowb[b], gsem[b]).wait()
        pltpu.make_async_copy(tdst_hbm.at[id_], drowb[b], gsem[b]).wait()

    zero16 = jnp.zeros((16,), jnp.float32)

    @pl.loop(0, ECH)
    def _(r):
        for k in range(TW // 16):
            rows0[r, pl.ds(16 * k, 16)] = zero16

    @pl.loop(0, RPS // ECH)
    def _(z):
        pltpu.sync_copy(rows0.at[pl.ds(0, ECH)],
                        acc.at[pl.ds(sid * RPS + z * ECH, ECH)])

    plsc.subcore_barrier()

    pltpu.sync_copy(m_hbm.at[0], mvec)
    m = mvec[...]
    pat = lax.rem(lax.iota(jnp.int32, 16), jnp.full((16,), 8, jnp.int32))
    colv = pat + jnp.full((16,), HC, jnp.int32)

    def compute(b):
        rows_, drows_ = rowb[b], drowb[b]

        @plsc.parallel_loop(0, ECH, unroll=8)
        def _(e):
            a_s = rows_[e, pl.ds(HC, 16)]
            a_d = drows_[e, pl.ds(0, 16)]
            t = a_s + a_d
            lrelu = jnp.maximum(t, 0.2 * t)
            wv = jnp.exp(lrelu - m)
            rows_[e, pl.ds(HC, 16)] = wv
            rowv = jnp.full((16,), e, jnp.int32)
            wt = plsc.load_gather(rows_, [rowv, colv])
            for k in range(H):
                sl = pl.ds(16 * k, 16)
                rows_[e, sl] = rows_[e, sl] * wt

    def scatter(b, q):
        pltpu.sync_copy(rowb[b], acc.at[idx[b][q][1]], add=True)

    for k in range(4):
        fire_idx(k, k % 2, (k // 2) % 2)
    for k in range(2):
        fire_gather(k % 2, 0)

    @pl.loop(0, (_BASE_CH - 1) // 4)
    def _(u):
        for r in range(4):
            k = 4 * u + r
            b = r % 2
            q = r // 2
            drain_gather(b, q)
            compute(b)
            scatter(b, q)

            @pl.when(k + 4 < _BASE_CH)
            def _():
                fire_idx(k + 4, b, q)

            @pl.when(k + 2 < _BASE_CH)
            def _():
                fire_gather(b, 1 - q)

    drain_gather(0, 0)
    compute(0)
    scatter(0, 0)

    plsc.subcore_barrier()

    @pl.loop(0, 5)
    def _(z):
        r0 = sid * RPS + z * (RPS // 5)
        pltpu.sync_copy(acc.at[pl.ds(r0, RPS // 5), pl.ds(0, HC)],
                        outm_hbm.at[cid, pl.ds(r0, RPS // 5)])
        pltpu.sync_copy(acc.at[pl.ds(r0, RPS // 5), pl.ds(HC, 16)],
                        outd_hbm.at[cid, pl.ds(r0, RPS // 5)])


_SC_CP = pltpu.CompilerParams(needs_layout_passes=False,
                              use_tc_tiling_on_sc=False)


def _sc_edges(tsrc, tdst, ei, m16):
    return pl.kernel(
        _sc_body,
        compiler_params=_SC_CP,
        out_type=[jax.ShapeDtypeStruct((2, NPAD, HC), jnp.float32),
                  jax.ShapeDtypeStruct((2, NPAD, 16), jnp.float32)],
        mesh=plsc.VectorSubcoreMesh(core_axis_name="c", subcore_axis_name="s"),
        scratch_types=(
            [pltpu.VMEM((ECH,), jnp.int32)] * 8
            + [
                pltpu.VMEM((ECH, TW), jnp.float32),
                pltpu.VMEM((ECH, 16), jnp.float32),
                pltpu.VMEM((ECH, TW), jnp.float32),
                pltpu.VMEM((ECH, 16), jnp.float32),
                pltpu.VMEM((16,), jnp.float32),
                pltpu.VMEM_SHARED((NPAD, TW), jnp.float32),
            ]
            + [pltpu.SemaphoreType.DMA] * 6
        ),
    )(tsrc, tdst, ei, m16)


def _final_body(pm_ref, pd_ref, tsrc_ref, tdst_ref, m_ref, b_ref, perm_ref,
                t8_ref, o_ref):
    tsrc = tsrc_ref[...]
    asrc = tsrc[:, HC:HC + 8]
    adst = tdst_ref[...][:, :8]
    t = asrc + adst
    lrelu = jnp.maximum(t, 0.2 * t)
    wself = jnp.exp(lrelu - m_ref[0, :8][None, :])
    t8 = t8_ref[...]
    num = pm_ref[0] + pm_ref[1] + tsrc[:, :HC] * jnp.dot(wself, t8)
    rden = 1.0 / (pd_ref[0][:, :8] + pd_ref[1][:, :8] + wself + 1e-16)
    outc = num * jnp.dot(rden, t8)
    o_ref[...] = jnp.dot(outc, perm_ref[...]) + b_ref[0][None, :]


def _final(pm, pd, tsrc, tdst, m16, bias, P, T8):
    return pl.pallas_call(
        _final_body,
        grid=(NBLK,),
        in_specs=[
            pl.BlockSpec((2, NB, HC), lambda i: (0, i, 0)),
            pl.BlockSpec((2, NB, 16), lambda i: (0, i, 0)),
            pl.BlockSpec((NB, TW), lambda i: (i, 0)),
            pl.BlockSpec((NB, 16), lambda i: (i, 0)),
            pl.BlockSpec((8, 16), lambda i: (0, 0)),
            pl.BlockSpec((1, HC), lambda i: (0, 0)),
            pl.BlockSpec((HC, HC), lambda i: (0, 0)),
            pl.BlockSpec((8, HC), lambda i: (0, 0)),
        ],
        out_specs=pl.BlockSpec((NB, HC), lambda i: (i, 0)),
        out_shape=jax.ShapeDtypeStruct((N, HC), jnp.float32),
    )(pm, pd, tsrc, tdst, m16, bias, P, T8)


def kernel(x, edge_index, W, att_src, att_dst, bias):
    idx = jnp.arange(HC, dtype=jnp.int32)
    cmajor_of = (idx % 8) * C + idx // 8
    Wp = W[:, cmajor_of]
    onehot = jax.nn.one_hot(idx % 8, 8, dtype=jnp.float32)
    A_s = att_src.reshape(H, C).T.reshape(HC, 1) * onehot
    A_d = att_dst.reshape(H, C).T.reshape(HC, 1) * onehot
    P = jax.nn.one_hot(cmajor_of, HC, dtype=jnp.float32)
    T8 = jax.nn.one_hot(idx % 8, 8, dtype=jnp.float32).T

    tsrc, tdst, _, m16 = _prep(x, Wp, A_s, A_d)
    pm, pd = _sc_edges(tsrc, tdst, edge_index, m16)
    return _final(pm, pd, tsrc, tdst, m16, bias.reshape(1, HC), P, T8)


# speedup vs baseline: 1.6926x; 1.0380x over previous
import jax
import jax.numpy as jnp
from jax import lax
from jax.experimental import pallas as pl
from jax.experimental.pallas import tpu as pltpu
from jax.experimental.pallas import tpu_sc as plsc

N = 10000
E = 320000
D = 128
H = 8
C = 16
HC = H * C
TW = HC + 16

NB = 2000
NBLK = N // NB

ECH = 80
NCHUNK = E // ECH
NWORK = 32
_BASE_CH = NCHUNK // NWORK
NPAD = 10240
RPS = NPAD // 16

_HI = lax.Precision.HIGHEST


def _prep_body(x_ref, wp_ref, as_ref, ad_ref, tsrc_ref, tdst_ref,
               pmax_ref, m16_ref):
    hc = jnp.dot(x_ref[...], wp_ref[...])
    a_s = jnp.dot(hc, as_ref[...])
    a_d = jnp.dot(hc, ad_ref[...])
    zeros8 = jnp.zeros((NB, 8), jnp.float32)
    tsrc_ref[...] = jnp.concatenate([hc, a_s, zeros8], axis=1)
    tdst_ref[...] = jnp.concatenate([a_d, zeros8], axis=1)
    cur = jnp.broadcast_to(
        jnp.concatenate([jnp.max(a_s, axis=0), jnp.max(a_d, axis=0)])[None, :],
        (8, 16))
    i = pl.program_id(0)

    @pl.when(i == 0)
    def _():
        pmax_ref[...] = cur

    @pl.when(i > 0)
    def _():
        pmax_ref[...] = jnp.maximum(pmax_ref[...], cur)

    @pl.when(i == NBLK - 1)
    def _():
        pm = pmax_ref[...]
        tot = pm + jnp.roll(pm, -8, axis=1)
        lane = lax.broadcasted_iota(jnp.int32, (8, 16), 1)
        m16_ref[...] = jnp.where(lane < 8, tot, 1e30)


def _prep(x, Wp, A_s, A_d):
    return pl.pallas_call(
        _prep_body,
        grid=(NBLK,),
        in_specs=[
            pl.BlockSpec((NB, D), lambda i: (i, 0)),
            pl.BlockSpec((D, HC), lambda i: (0, 0)),
            pl.BlockSpec((HC, 8), lambda i: (0, 0)),
            pl.BlockSpec((HC, 8), lambda i: (0, 0)),
        ],
        out_specs=[
            pl.BlockSpec((NB, TW), lambda i: (i, 0)),
            pl.BlockSpec((NB, 16), lambda i: (i, 0)),
            pl.BlockSpec((8, 16), lambda i: (0, 0)),
            pl.BlockSpec((8, 16), lambda i: (0, 0)),
        ],
        out_shape=[
            jax.ShapeDtypeStruct((N, TW), jnp.float32),
            jax.ShapeDtypeStruct((N, 16), jnp.float32),
            jax.ShapeDtypeStruct((8, 16), jnp.float32),
            jax.ShapeDtypeStruct((8, 16), jnp.float32),
        ],
    )(x, Wp, A_s, A_d)


def _sc_body(tsrc_hbm, tdst_hbm, ei_hbm, m_hbm, outm_hbm, outd_hbm,
             is00, id00, is01, id01, is10, id10, is11, id11,
             rows0, drows0, rows1, drows1, mvec, acc,
             gsem0, gsem1, isem00, isem01, isem10, isem11):
    cid = lax.axis_index("c")
    sid = lax.axis_index("s")
    wid = sid * 2 + cid
    base = wid * _BASE_CH * ECH

    rowb = (rows0, rows1)
    drowb = (drows0, drows1)
    gsem = (gsem0, gsem1)
    idx = (((is00, id00, isem00), (is01, id01, isem01)),
           ((is10, id10, isem10), (is11, id11, isem11)))

    def fire_idx(k, b, q):
        is_, id_, sem = idx[b][q]
        off = base + k * ECH
        pltpu.async_copy(ei_hbm.at[0, pl.ds(off, ECH)], is_, sem)
        pltpu.async_copy(ei_hbm.at[1, pl.ds(off, ECH)], id_, sem)

    def fire_gather(b, q):
        is_, id_, sem = idx[b][q]
        pltpu.make_async_copy(ei_hbm.at[0, pl.ds(0, ECH)], is_, sem).wait()
        pltpu.make_async_copy(ei_hbm.at[1, pl.ds(0, ECH)], id_, sem).wait()
        pltpu.async_copy(tsrc_hbm.at[is_], rowb[b], gsem[b])
        pltpu.async_copy(tdst_hbm.at[id_], drowb[b], gsem[b])

    def drain_gather(b, q):
        is_, id_, _ = idx[b][q]
        pltpu.make_async_copy(tsrc_hbm.at[is_], rowb[b], gsem[b]).wait()
        pltpu.make_async_copy(tdst_hbm.at[id_], drowb[b], gsem[b]).wait()

    zero16 = jnp.zeros((16,), jnp.float32)

    @pl.loop(0, ECH)
    def _(r):
        for k in range(TW // 16):
            rows0[r, pl.ds(16 * k, 16)] = zero16

    for z in range(RPS // ECH):
        pltpu.async_copy(rows0.at[pl.ds(0, ECH)],
                         acc.at[pl.ds(sid * RPS + z * ECH, ECH)], gsem0)
    for z in range(RPS // ECH):
        pltpu.make_async_copy(rows0.at[pl.ds(0, ECH)],
                              acc.at[pl.ds(sid * RPS + z * ECH, ECH)],
                              gsem0).wait()

    plsc.subcore_barrier()

    pltpu.sync_copy(m_hbm.at[0], mvec)
    m = mvec[...]
    pat = lax.rem(lax.iota(jnp.int32, 16), jnp.full((16,), 8, jnp.int32))
    colv = pat + jnp.full((16,), HC, jnp.int32)

    def compute(b):
        rows_, drows_ = rowb[b], drowb[b]

        @plsc.parallel_loop(0, ECH, unroll=8)
        def _(e):
            a_s = rows_[e, pl.ds(HC, 16)]
            a_d = drows_[e, pl.ds(0, 16)]
            t = a_s + a_d
            lrelu = jnp.maximum(t, 0.2 * t)
            wv = jnp.exp(lrelu - m)
            rows_[e, pl.ds(HC, 16)] = wv
            rowv = jnp.full((16,), e, jnp.int32)
            wt = plsc.load_gather(rows_, [rowv, colv])
            for k in range(H):
                sl = pl.ds(16 * k, 16)
                rows_[e, sl] = rows_[e, sl] * wt

    def scatter(b, q):
        pltpu.sync_copy(rowb[b], acc.at[idx[b][q][1]], add=True)

    for k in range(4):
        fire_idx(k, k % 2, (k // 2) % 2)
    for k in range(2):
        fire_gather(k % 2, 0)

    @pl.loop(0, (_BASE_CH - 1) // 4)
    def _(u):
        for r in range(4):
            k = 4 * u + r
            b = r % 2
            q = r // 2
            drain_gather(b, q)
            compute(b)
            scatter(b, q)

            @pl.when(k + 4 < _BASE_CH)
            def _():
                fire_idx(k + 4, b, q)

            @pl.when(k + 2 < _BASE_CH)
            def _():
                fire_gather(b, 1 - q)

    drain_gather(0, 0)
    compute(0)
    scatter(0, 0)

    plsc.subcore_barrier()

    for z in range(5):
        r0 = sid * RPS + z * (RPS // 5)
        pltpu.async_copy(acc.at[pl.ds(r0, RPS // 5), pl.ds(0, HC)],
                         outm_hbm.at[cid, pl.ds(r0, RPS // 5)], gsem0)
        pltpu.async_copy(acc.at[pl.ds(r0, RPS // 5), pl.ds(HC, 16)],
                         outd_hbm.at[cid, pl.ds(r0, RPS // 5)], gsem1)
    for z in range(5):
        r0 = sid * RPS + z * (RPS // 5)
        pltpu.make_async_copy(acc.at[pl.ds(r0, RPS // 5), pl.ds(0, HC)],
                              outm_hbm.at[cid, pl.ds(r0, RPS // 5)],
                              gsem0).wait()
        pltpu.make_async_copy(acc.at[pl.ds(r0, RPS // 5), pl.ds(HC, 16)],
                              outd_hbm.at[cid, pl.ds(r0, RPS // 5)],
                              gsem1).wait()


_SC_CP = pltpu.CompilerParams(needs_layout_passes=False,
                              use_tc_tiling_on_sc=False)


def _sc_edges(tsrc, tdst, ei, m16):
    return pl.kernel(
        _sc_body,
        compiler_params=_SC_CP,
        out_type=[jax.ShapeDtypeStruct((2, NPAD, HC), jnp.float32),
                  jax.ShapeDtypeStruct((2, NPAD, 16), jnp.float32)],
        mesh=plsc.VectorSubcoreMesh(core_axis_name="c", subcore_axis_name="s"),
        scratch_types=(
            [pltpu.VMEM((ECH,), jnp.int32)] * 8
            + [
                pltpu.VMEM((ECH, TW), jnp.float32),
                pltpu.VMEM((ECH, 16), jnp.float32),
                pltpu.VMEM((ECH, TW), jnp.float32),
                pltpu.VMEM((ECH, 16), jnp.float32),
                pltpu.VMEM((16,), jnp.float32),
                pltpu.VMEM_SHARED((NPAD, TW), jnp.float32),
            ]
            + [pltpu.SemaphoreType.DMA] * 6
        ),
    )(tsrc, tdst, ei, m16)


def _final_body(pm_ref, pd_ref, tsrc_ref, tdst_ref, m_ref, b_ref, perm_ref,
                t8_ref, o_ref):
    tsrc = tsrc_ref[...]
    asrc = tsrc[:, HC:HC + 8]
    adst = tdst_ref[...][:, :8]
    t = asrc + adst
    lrelu = jnp.maximum(t, 0.2 * t)
    wself = jnp.exp(lrelu - m_ref[0, :8][None, :])
    t8 = t8_ref[...]
    num = pm_ref[0] + pm_ref[1] + tsrc[:, :HC] * jnp.dot(wself, t8)
    rden = 1.0 / (pd_ref[0][:, :8] + pd_ref[1][:, :8] + wself + 1e-16)
    outc = num * jnp.dot(rden, t8)
    o_ref[...] = jnp.dot(outc, perm_ref[...]) + b_ref[0][None, :]


def _final(pm, pd, tsrc, tdst, m16, bias, P, T8):
    return pl.pallas_call(
        _final_body,
        grid=(NBLK,),
        in_specs=[
            pl.BlockSpec((2, NB, HC), lambda i: (0, i, 0)),
            pl.BlockSpec((2, NB, 16), lambda i: (0, i, 0)),
            pl.BlockSpec((NB, TW), lambda i: (i, 0)),
            pl.BlockSpec((NB, 16), lambda i: (i, 0)),
            pl.BlockSpec((8, 16), lambda i: (0, 0)),
            pl.BlockSpec((1, HC), lambda i: (0, 0)),
            pl.BlockSpec((HC, HC), lambda i: (0, 0)),
            pl.BlockSpec((8, HC), lambda i: (0, 0)),
        ],
        out_specs=pl.BlockSpec((NB, HC), lambda i: (i, 0)),
        out_shape=jax.ShapeDtypeStruct((N, HC), jnp.float32),
    )(pm, pd, tsrc, tdst, m16, bias, P, T8)


def kernel(x, edge_index, W, att_src, att_dst, bias):
    idx = jnp.arange(HC, dtype=jnp.int32)
    cmajor_of = (idx % 8) * C + idx // 8
    Wp = W[:, cmajor_of]
    onehot = jax.nn.one_hot(idx % 8, 8, dtype=jnp.float32)
    A_s = att_src.reshape(H, C).T.reshape(HC, 1) * onehot
    A_d = att_dst.reshape(H, C).T.reshape(HC, 1) * onehot
    P = jax.nn.one_hot(cmajor_of, HC, dtype=jnp.float32)
    T8 = jax.nn.one_hot(idx % 8, 8, dtype=jnp.float32).T

    tsrc, tdst, _, m16 = _prep(x, Wp, A_s, A_d)
    pm, pd = _sc_edges(tsrc, tdst, edge_index, m16)
    return _final(pm, pd, tsrc, tdst, m16, bias.reshape(1, HC), P, T8)


# speedup vs baseline: 1.7066x; 1.0083x over previous
import jax
import jax.numpy as jnp
from jax import lax
from jax.experimental import pallas as pl
from jax.experimental.pallas import tpu as pltpu
from jax.experimental.pallas import tpu_sc as plsc

N = 10000
E = 320000
D = 128
H = 8
C = 16
HC = H * C
TW = HC + 16

NB = 2000
NBLK = N // NB

ECH = 80
NCHUNK = E // ECH
NWORK = 32
_BASE_CH = NCHUNK // NWORK
NPAD = 10240
RPS = NPAD // 16

_HI = lax.Precision.HIGHEST


def _prep_body(x_ref, wp_ref, as_ref, ad_ref, tsrc_ref, tdst_ref,
               pmax_ref, m16_ref):
    hc = jnp.dot(x_ref[...], wp_ref[...])
    a_s = jnp.dot(hc, as_ref[...])
    a_d = jnp.dot(hc, ad_ref[...])
    zeros8 = jnp.zeros((NB, 8), jnp.float32)
    tsrc_ref[...] = jnp.concatenate([hc, a_s, zeros8], axis=1)
    tdst_ref[...] = jnp.concatenate([a_d, zeros8], axis=1)
    cur = jnp.broadcast_to(
        jnp.concatenate([jnp.max(a_s, axis=0), jnp.max(a_d, axis=0)])[None, :],
        (8, 16))
    i = pl.program_id(0)

    @pl.when(i == 0)
    def _():
        pmax_ref[...] = cur

    @pl.when(i > 0)
    def _():
        pmax_ref[...] = jnp.maximum(pmax_ref[...], cur)

    @pl.when(i == NBLK - 1)
    def _():
        pm = pmax_ref[...]
        tot = pm + jnp.roll(pm, -8, axis=1)
        lane = lax.broadcasted_iota(jnp.int32, (8, 16), 1)
        m16_ref[...] = jnp.where(lane < 8, tot, 1e30)


def _prep(x, Wp, A_s, A_d):
    return pl.pallas_call(
        _prep_body,
        grid=(NBLK,),
        in_specs=[
            pl.BlockSpec((NB, D), lambda i: (i, 0)),
            pl.BlockSpec((D, HC), lambda i: (0, 0)),
            pl.BlockSpec((HC, 8), lambda i: (0, 0)),
            pl.BlockSpec((HC, 8), lambda i: (0, 0)),
        ],
        out_specs=[
            pl.BlockSpec((NB, TW), lambda i: (i, 0)),
            pl.BlockSpec((NB, 16), lambda i: (i, 0)),
            pl.BlockSpec((8, 16), lambda i: (0, 0)),
            pl.BlockSpec((8, 16), lambda i: (0, 0)),
        ],
        out_shape=[
            jax.ShapeDtypeStruct((N, TW), jnp.float32),
            jax.ShapeDtypeStruct((N, 16), jnp.float32),
            jax.ShapeDtypeStruct((8, 16), jnp.float32),
            jax.ShapeDtypeStruct((8, 16), jnp.float32),
        ],
    )(x, Wp, A_s, A_d)


def _sc_body(tsrc_hbm, tdst_hbm, ei_hbm, m_hbm, outm_hbm, outd_hbm,
             is00, id00, is01, id01, is10, id10, is11, id11,
             rows0, drows0, rows1, drows1, mvec, acc,
             gsem0, gsem1, isem00, isem01, isem10, isem11):
    cid = lax.axis_index("c")
    sid = lax.axis_index("s")
    wid = sid * 2 + cid
    base = wid * _BASE_CH * ECH

    rowb = (rows0, rows1)
    drowb = (drows0, drows1)
    gsem = (gsem0, gsem1)
    idx = (((is00, id00, isem00), (is01, id01, isem01)),
           ((is10, id10, isem10), (is11, id11, isem11)))

    def fire_idx(k, b, q):
        is_, id_, sem = idx[b][q]
        off = base + k * ECH
        pltpu.async_copy(ei_hbm.at[0, pl.ds(off, ECH)], is_, sem)
        pltpu.async_copy(ei_hbm.at[1, pl.ds(off, ECH)], id_, sem)

    def fire_gather(b, q):
        is_, id_, sem = idx[b][q]
        pltpu.make_async_copy(ei_hbm.at[0, pl.ds(0, ECH)], is_, sem).wait()
        pltpu.make_async_copy(ei_hbm.at[1, pl.ds(0, ECH)], id_, sem).wait()
        pltpu.async_copy(tsrc_hbm.at[is_], rowb[b], gsem[b])
        pltpu.async_copy(tdst_hbm.at[id_], drowb[b], gsem[b])

    def drain_gather(b, q):
        is_, id_, _ = idx[b][q]
        pltpu.make_async_copy(tsrc_hbm.at[is_], rowb[b], gsem[b]).wait()
        pltpu.make_async_copy(tdst_hbm.at[id_], drowb[b], gsem[b]).wait()

    zero16 = jnp.zeros((16,), jnp.float32)

    @pl.loop(0, ECH)
    def _(r):
        for k in range(TW // 16):
            rows1[r, pl.ds(16 * k, 16)] = zero16

    for z in range(RPS // ECH):
        pltpu.async_copy(rows1.at[pl.ds(0, ECH)],
                         acc.at[pl.ds(sid * RPS + z * ECH, ECH)], gsem1)
    for k in range(4):
        fire_idx(k, k % 2, (k // 2) % 2)
    fire_gather(0, 0)
    for z in range(RPS // ECH):
        pltpu.make_async_copy(rows1.at[pl.ds(0, ECH)],
                              acc.at[pl.ds(sid * RPS + z * ECH, ECH)],
                              gsem1).wait()
    fire_gather(1, 0)

    plsc.subcore_barrier()

    pltpu.sync_copy(m_hbm.at[0], mvec)
    m = mvec[...]
    pat = lax.rem(lax.iota(jnp.int32, 16), jnp.full((16,), 8, jnp.int32))
    colv = pat + jnp.full((16,), HC, jnp.int32)

    def compute(b):
        rows_, drows_ = rowb[b], drowb[b]

        @plsc.parallel_loop(0, ECH, unroll=8)
        def _(e):
            a_s = rows_[e, pl.ds(HC, 16)]
            a_d = drows_[e, pl.ds(0, 16)]
            t = a_s + a_d
            lrelu = jnp.maximum(t, 0.2 * t)
            wv = jnp.exp(lrelu - m)
            rows_[e, pl.ds(HC, 16)] = wv
            rowv = jnp.full((16,), e, jnp.int32)
            wt = plsc.load_gather(rows_, [rowv, colv])
            for k in range(H):
                sl = pl.ds(16 * k, 16)
                rows_[e, sl] = rows_[e, sl] * wt

    def scatter(b, q):
        pltpu.sync_copy(rowb[b], acc.at[idx[b][q][1]], add=True)

    @pl.loop(0, (_BASE_CH - 1) // 4)
    def _(u):
        for r in range(4):
            k = 4 * u + r
            b = r % 2
            q = r // 2
            drain_gather(b, q)
            compute(b)
            scatter(b, q)

            @pl.when(k + 4 < _BASE_CH)
            def _():
                fire_idx(k + 4, b, q)

            @pl.when(k + 2 < _BASE_CH)
            def _():
                fire_gather(b, 1 - q)

    drain_gather(0, 0)
    compute(0)
    scatter(0, 0)

    plsc.subcore_barrier()

    for z in range(5):
        r0 = sid * RPS + z * (RPS // 5)
        pltpu.async_copy(acc.at[pl.ds(r0, RPS // 5), pl.ds(0, HC)],
                         outm_hbm.at[cid, pl.ds(r0, RPS // 5)], gsem0)
        pltpu.async_copy(acc.at[pl.ds(r0, RPS // 5), pl.ds(HC, 16)],
                         outd_hbm.at[cid, pl.ds(r0, RPS // 5)], gsem1)
    for z in range(5):
        r0 = sid * RPS + z * (RPS // 5)
        pltpu.make_async_copy(acc.at[pl.ds(r0, RPS // 5), pl.ds(0, HC)],
                              outm_hbm.at[cid, pl.ds(r0, RPS // 5)],
                              gsem0).wait()
        pltpu.make_async_copy(acc.at[pl.ds(r0, RPS // 5), pl.ds(HC, 16)],
                              outd_hbm.at[cid, pl.ds(r0, RPS // 5)],
                              gsem1).wait()


_SC_CP = pltpu.CompilerParams(needs_layout_passes=False,
                              use_tc_tiling_on_sc=False)


def _sc_edges(tsrc, tdst, ei, m16):
    return pl.kernel(
        _sc_body,
        compiler_params=_SC_CP,
        out_type=[jax.ShapeDtypeStruct((2, NPAD, HC), jnp.float32),
                  jax.ShapeDtypeStruct((2, NPAD, 16), jnp.float32)],
        mesh=plsc.VectorSubcoreMesh(core_axis_name="c", subcore_axis_name="s"),
        scratch_types=(
            [pltpu.VMEM((ECH,), jnp.int32)] * 8
            + [
                pltpu.VMEM((ECH, TW), jnp.float32),
                pltpu.VMEM((ECH, 16), jnp.float32),
                pltpu.VMEM((ECH, TW), jnp.float32),
                pltpu.VMEM((ECH, 16), jnp.float32),
                pltpu.VMEM((16,), jnp.float32),
                pltpu.VMEM_SHARED((NPAD, TW), jnp.float32),
            ]
            + [pltpu.SemaphoreType.DMA] * 6
        ),
    )(tsrc, tdst, ei, m16)


def _final_body(pm_ref, pd_ref, tsrc_ref, tdst_ref, m_ref, b_ref, perm_ref,
                t8_ref, o_ref):
    tsrc = tsrc_ref[...]
    asrc = tsrc[:, HC:HC + 8]
    adst = tdst_ref[...][:, :8]
    t = asrc + adst
    lrelu = jnp.maximum(t, 0.2 * t)
    wself = jnp.exp(lrelu - m_ref[0, :8][None, :])
    t8 = t8_ref[...]
    num = pm_ref[0] + pm_ref[1] + tsrc[:, :HC] * jnp.dot(wself, t8)
    rden = 1.0 / (pd_ref[0][:, :8] + pd_ref[1][:, :8] + wself + 1e-16)
    outc = num * jnp.dot(rden, t8)
    o_ref[...] = jnp.dot(outc, perm_ref[...]) + b_ref[0][None, :]


def _final(pm, pd, tsrc, tdst, m16, bias, P, T8):
    return pl.pallas_call(
        _final_body,
        grid=(NBLK,),
        in_specs=[
            pl.BlockSpec((2, NB, HC), lambda i: (0, i, 0)),
            pl.BlockSpec((2, NB, 16), lambda i: (0, i, 0)),
            pl.BlockSpec((NB, TW), lambda i: (i, 0)),
            pl.BlockSpec((NB, 16), lambda i: (i, 0)),
            pl.BlockSpec((8, 16), lambda i: (0, 0)),
            pl.BlockSpec((1, HC), lambda i: (0, 0)),
            pl.BlockSpec((HC, HC), lambda i: (0, 0)),
            pl.BlockSpec((8, HC), lambda i: (0, 0)),
        ],
        out_specs=pl.BlockSpec((NB, HC), lambda i: (i, 0)),
        out_shape=jax.ShapeDtypeStruct((N, HC), jnp.float32),
    )(pm, pd, tsrc, tdst, m16, bias, P, T8)


def kernel(x, edge_index, W, att_src, att_dst, bias):
    idx = jnp.arange(HC, dtype=jnp.int32)
    cmajor_of = (idx % 8) * C + idx // 8
    Wp = W[:, cmajor_of]
    onehot = jax.nn.one_hot(idx % 8, 8, dtype=jnp.float32)
    A_s = att_src.reshape(H, C).T.reshape(HC, 1) * onehot
    A_d = att_dst.reshape(H, C).T.reshape(HC, 1) * onehot
    P = jax.nn.one_hot(cmajor_of, HC, dtype=jnp.float32)
    T8 = jax.nn.one_hot(idx % 8, 8, dtype=jnp.float32).T

    tsrc, tdst, _, m16 = _prep(x, Wp, A_s, A_d)
    pm, pd = _sc_edges(tsrc, tdst, edge_index, m16)
    return _final(pm, pd, tsrc, tdst, m16, bias.reshape(1, HC), P, T8)
